# initial kernel scaffold (unmeasured)
import jax
import jax.numpy as jnp
from jax import lax
from jax.experimental import pallas as pl
from jax.experimental.pallas import tpu as pltpu

N_DEV = 8
BLK = 64


def kernel(x, Wq, K_ext, V_ext, Wo):
    B, SQ, E = x.shape
    _, SKV, HQ, DH = K_ext.shape
    D = HQ * DH
    n_hops = N_DEV - 1

    def body(x_ref, wq_ref, k_ref, v_ref, wo_ref, out_ref,
             kc_ref, vc_ref, ksend, krecv, vsend, vrecv):
        my = lax.axis_index("i")
        left = lax.rem(my + N_DEV - 1, N_DEV)
        right = lax.rem(my + 1, N_DEV)

        barrier_sem = pltpu.get_barrier_semaphore()
        for nbr in (left, right):
            pl.semaphore_signal(
                barrier_sem, inc=1,
                device_id=(nbr,), device_id_type=pl.DeviceIdType.MESH,
            )
        pl.semaphore_wait(barrier_sem, 2)

        q = [jnp.dot(x_ref[b, :, :], wq_ref[:, :],
                     preferred_element_type=jnp.float32) * 0.125
             for b in range(B)]

        acc = [[jnp.zeros((SQ, DH), jnp.float32) for _ in range(HQ)]
               for _ in range(B)]
        den = [[jnp.zeros((SQ, 1), jnp.float32) for _ in range(HQ)]
               for _ in range(B)]

        qb = my * (SQ // BLK) + lax.broadcasted_iota(
            jnp.int32, (SQ, SKV), 0) // BLK
        kv_col = lax.broadcasted_iota(jnp.int32, (SQ, SKV), 1) // BLK

        for h in range(N_DEV):
            if h < n_hops:
                src_k = k_ref if h == 0 else kc_ref.at[h - 1]
                src_v = v_ref if h == 0 else vc_ref.at[h - 1]
                rk = pltpu.make_async_remote_copy(
                    src_ref=src_k, dst_ref=kc_ref.at[h],
                    send_sem=ksend.at[h], recv_sem=krecv.at[h],
                    device_id=(right,), device_id_type=pl.DeviceIdType.MESH,
                )
                rv = pltpu.make_async_remote_copy(
                    src_ref=src_v, dst_ref=vc_ref.at[h],
                    send_sem=vsend.at[h], recv_sem=vrecv.at[h],
                    device_id=(right,), device_id_type=pl.DeviceIdType.MESH,
                )
                rk.start()
                rv.start()

            origin = lax.rem(my - h + N_DEV, N_DEV)
            kb = origin * (SKV // BLK) + kv_col
            mask = (qb == kb) | (kb == 0) | (lax.rem(qb + kb, 3) == 0)

            for b in range(B):
                kc = (k_ref[b, :, :, :] if h == 0
                      else kc_ref[h - 1, b, :, :, :]).reshape(SKV, D)
                vc = (v_ref[b, :, :, :] if h == 0
                      else vc_ref[h - 1, b, :, :, :]).reshape(SKV, D)
                for hh in range(HQ):
                    qbh = q[b][:, hh * DH:(hh + 1) * DH]
                    khh = kc[:, hh * DH:(hh + 1) * DH]
                    vhh = vc[:, hh * DH:(hh + 1) * DH]
                    s = lax.dot_general(
                        qbh, khh, (((1,), (1,)), ((), ())),
                        preferred_element_type=jnp.float32)
                    w = jnp.where(mask, jnp.exp(s), 0.0)
                    acc[b][hh] = acc[b][hh] + jnp.dot(
                        w, vhh, preferred_element_type=jnp.float32)
                    den[b][hh] = den[b][hh] + jnp.sum(w, axis=1, keepdims=True)

            if h < n_hops:
                rk.wait()
                rv.wait()

        for b in range(B):
            ctx = jnp.concatenate(
                [acc[b][hh] / den[b][hh] for hh in range(HQ)], axis=1)
            out_ref[b, :, :] = jnp.dot(
                ctx, wo_ref[:, :], preferred_element_type=jnp.float32)

    return pl.pallas_call(
        body,
        out_shape=jax.ShapeDtypeStruct((B, SQ, E), jnp.float32),
        in_specs=[pl.BlockSpec(memory_space=pltpu.VMEM)] * 5,
        out_specs=pl.BlockSpec(memory_space=pltpu.VMEM),
        scratch_shapes=[
            pltpu.VMEM((n_hops, B, SKV, HQ, DH), jnp.float32),
            pltpu.VMEM((n_hops, B, SKV, HQ, DH), jnp.float32),
            pltpu.SemaphoreType.DMA((n_hops,)),
            pltpu.SemaphoreType.DMA((n_hops,)),
            pltpu.SemaphoreType.DMA((n_hops,)),
            pltpu.SemaphoreType.DMA((n_hops,)),
        ],
        compiler_params=pltpu.CompilerParams(collective_id=0),
    )(x, Wq, K_ext, V_ext, Wo)


# baseline (device time: 389524 ns/iter reference)
import jax
import jax.numpy as jnp
from jax import lax
from jax.experimental import pallas as pl
from jax.experimental.pallas import tpu as pltpu

N_DEV = 8
BLK = 64


def kernel(x, Wq, K_ext, V_ext, Wo):
    B, SQ, E = x.shape
    _, SKV, HQ, DH = K_ext.shape
    D = HQ * DH
    n_hops = N_DEV - 1

    def body(x_ref, wq_ref, k_ref, v_ref, wo_ref, out_ref,
             kc_ref, vc_ref, q_ref, acc_ref,
             ksend, krecv, vsend, vrecv):
        my = lax.axis_index("i")
        left = lax.rem(my + N_DEV - 1, N_DEV)
        right = lax.rem(my + 1, N_DEV)

        barrier_sem = pltpu.get_barrier_semaphore()
        for nbr in (left, right):
            pl.semaphore_signal(
                barrier_sem, inc=1,
                device_id=(nbr,), device_id_type=pl.DeviceIdType.MESH,
            )
        pl.semaphore_wait(barrier_sem, 2)

        for b in range(B):
            q_ref[b, :, :] = jnp.dot(
                x_ref[b, :, :], wq_ref[:, :],
                preferred_element_type=jnp.float32) * 0.125
            kc_ref[0, b, :, :] = k_ref[b, :, :, :].reshape(SKV, D)
            vc_ref[0, b, :, :] = v_ref[b, :, :, :].reshape(SKV, D)

        qb = my * (SQ // BLK) + lax.broadcasted_iota(
            jnp.int32, (SQ, SKV), 0) // BLK
        kv_col = lax.broadcasted_iota(jnp.int32, (SQ, SKV), 1) // BLK
        ones_col = jnp.ones((SKV, 1), jnp.float32)

        for h in range(N_DEV):
            slot = h % 2
            nxt = 1 - slot

            origin = lax.rem(my - h + N_DEV, N_DEV)
            kb = origin * (SKV // BLK) + kv_col
            mask = (qb == kb) | (kb == 0) | (lax.rem(qb + kb, 3) == 0)

            for b in range(B):
                kcb = kc_ref[slot, b, :, :]
                vcb = vc_ref[slot, b, :, :]
                for hh in range(HQ):
                    qbh = q_ref[b, :, hh * DH:(hh + 1) * DH]
                    khh = kcb[:, hh * DH:(hh + 1) * DH]
                    s = lax.dot_general(
                        qbh, khh, (((1,), (1,)), ((), ())),
                        preferred_element_type=jnp.float32)
                    w = jnp.where(mask, jnp.exp(s), 0.0)
                    v_ext = jnp.concatenate(
                        [vcb[:, hh * DH:(hh + 1) * DH], ones_col], axis=1)
                    delta = jnp.dot(
                        w, v_ext, preferred_element_type=jnp.float32)
                    if h == 0:
                        acc_ref[b, hh, :, :] = delta
                    else:
                        acc_ref[b, hh, :, :] = acc_ref[b, hh, :, :] + delta

            if h < n_hops:
                rk = pltpu.make_async_remote_copy(
                    src_ref=kc_ref.at[slot], dst_ref=kc_ref.at[nxt],
                    send_sem=ksend.at[h], recv_sem=krecv.at[h],
                    device_id=(right,), device_id_type=pl.DeviceIdType.MESH,
                )
                rv = pltpu.make_async_remote_copy(
                    src_ref=vc_ref.at[slot], dst_ref=vc_ref.at[nxt],
                    send_sem=vsend.at[h], recv_sem=vrecv.at[h],
                    device_id=(right,), device_id_type=pl.DeviceIdType.MESH,
                )
                rk.start()
                rv.start()
                rk.wait()
                rv.wait()

        for b in range(B):
            ctx = jnp.concatenate(
                [acc_ref[b, hh, :, 0:DH] / acc_ref[b, hh, :, DH:DH + 1]
                 for hh in range(HQ)], axis=1)
            out_ref[b, :, :] = jnp.dot(
                ctx, wo_ref[:, :], preferred_element_type=jnp.float32)

    return pl.pallas_call(
        body,
        out_shape=jax.ShapeDtypeStruct((B, SQ, E), jnp.float32),
        in_specs=[pl.BlockSpec(memory_space=pltpu.VMEM)] * 5,
        out_specs=pl.BlockSpec(memory_space=pltpu.VMEM),
        scratch_shapes=[
            pltpu.VMEM((2, B, SKV, D), jnp.float32),
            pltpu.VMEM((2, B, SKV, D), jnp.float32),
            pltpu.VMEM((B, SQ, D), jnp.float32),
            pltpu.VMEM((B, HQ, SQ, DH + 1), jnp.float32),
            pltpu.SemaphoreType.DMA((n_hops,)),
            pltpu.SemaphoreType.DMA((n_hops,)),
            pltpu.SemaphoreType.DMA((n_hops,)),
            pltpu.SemaphoreType.DMA((n_hops,)),
        ],
        compiler_params=pltpu.CompilerParams(collective_id=0),
    )(x, Wq, K_ext, V_ext, Wo)


# device time: 208709 ns/iter; 1.8663x vs baseline; 1.8663x over previous
import jax
import jax.numpy as jnp
from jax import lax
from jax.experimental import pallas as pl
from jax.experimental.pallas import tpu as pltpu

N_DEV = 8
BLK = 64
R_HOPS = 3
L_HOPS = 4


def kernel(x, Wq, K_ext, V_ext, Wo):
    B, SQ, E = x.shape
    _, SKV, HQ, DH = K_ext.shape
    D = HQ * DH

    def body(x_ref, wq_ref, k_ref, v_ref, wo_ref, out_ref,
             kl_ref, vl_ref, kcr_ref, vcr_ref, kcl_ref, vcl_ref,
             q_ref, acc_ref,
             ksr, krr, vsr, vrr, ksl, krl, vsl, vrl):
        my = lax.axis_index("i")
        left = lax.rem(my + N_DEV - 1, N_DEV)
        right = lax.rem(my + 1, N_DEV)

        barrier_sem = pltpu.get_barrier_semaphore()
        for nbr in (left, right):
            pl.semaphore_signal(
                barrier_sem, inc=1,
                device_id=(nbr,), device_id_type=pl.DeviceIdType.MESH,
            )
        pl.semaphore_wait(barrier_sem, 2)

        for b in range(B):
            q_ref[b, :, :] = jnp.dot(
                x_ref[b, :, :], wq_ref[:, :],
                preferred_element_type=jnp.float32) * 0.125
            kl_ref[b, :, :] = k_ref[b, :, :, :].reshape(SKV, D)
            vl_ref[b, :, :] = v_ref[b, :, :, :].reshape(SKV, D)

        qb = my * (SQ // BLK) + lax.broadcasted_iota(
            jnp.int32, (SQ, SKV), 0) // BLK
        kv_col = lax.broadcasted_iota(jnp.int32, (SQ, SKV), 1) // BLK
        ones_col = jnp.ones((SKV, 1), jnp.float32)

        def attend(origin, kget, vget, first):
            kb = origin * (SKV // BLK) + kv_col
            mask = (qb == kb) | (kb == 0) | (lax.rem(qb + kb, 3) == 0)
            for b in range(B):
                kcb = kget(b)
                vcb = vget(b)
                for hh in range(HQ):
                    qbh = q_ref[b, :, hh * DH:(hh + 1) * DH]
                    khh = kcb[:, hh * DH:(hh + 1) * DH]
                    s = lax.dot_general(
                        qbh, khh, (((1,), (1,)), ((), ())),
                        preferred_element_type=jnp.float32)
                    w = jnp.where(mask, jnp.exp(s), 0.0)
                    v_ext = jnp.concatenate(
                        [vcb[:, hh * DH:(hh + 1) * DH], ones_col], axis=1)
                    delta = jnp.dot(
                        w, v_ext, preferred_element_type=jnp.float32)
                    if first:
                        acc_ref[b, hh, :, :] = delta
                    else:
                        acc_ref[b, hh, :, :] = acc_ref[b, hh, :, :] + delta

        def rdma(src, dst, ssem, rsem, dev):
            return pltpu.make_async_remote_copy(
                src_ref=src, dst_ref=dst, send_sem=ssem, recv_sem=rsem,
                device_id=(dev,), device_id_type=pl.DeviceIdType.MESH,
            )

        for s in range(L_HOPS + 1):
            waits = []
            if s < R_HOPS:
                ksrc = kl_ref if s == 0 else kcr_ref.at[(s - 1) % 2]
                vsrc = vl_ref if s == 0 else vcr_ref.at[(s - 1) % 2]
                rkr = rdma(ksrc, kcr_ref.at[s % 2], ksr.at[s], krr.at[s], right)
                rvr = rdma(vsrc, vcr_ref.at[s % 2], vsr.at[s], vrr.at[s], right)
                rkr.start()
                rvr.start()
                waits += [rkr, rvr]
            if s < L_HOPS:
                ksrc = kl_ref if s == 0 else kcl_ref.at[(s - 1) % 2]
                vsrc = vl_ref if s == 0 else vcl_ref.at[(s - 1) % 2]
                rkl = rdma(ksrc, kcl_ref.at[s % 2], ksl.at[s], krl.at[s], left)
                rvl = rdma(vsrc, vcl_ref.at[s % 2], vsl.at[s], vrl.at[s], left)
                rkl.start()
                rvl.start()
                waits += [rkl, rvl]

            if s == 0:
                attend(my,
                       lambda b: kl_ref[b, :, :],
                       lambda b: vl_ref[b, :, :], first=True)
            else:
                slot = (s - 1) % 2
                if s <= R_HOPS:
                    attend(lax.rem(my - s + N_DEV, N_DEV),
                           lambda b: kcr_ref[slot, b, :, :],
                           lambda b: vcr_ref[slot, b, :, :], first=False)
                if s <= L_HOPS:
                    attend(lax.rem(my + s, N_DEV),
                           lambda b: kcl_ref[slot, b, :, :],
                           lambda b: vcl_ref[slot, b, :, :], first=False)

            for r in waits:
                r.wait()

        for b in range(B):
            ctx = jnp.concatenate(
                [acc_ref[b, hh, :, 0:DH] / acc_ref[b, hh, :, DH:DH + 1]
                 for hh in range(HQ)], axis=1)
            out_ref[b, :, :] = jnp.dot(
                ctx, wo_ref[:, :], preferred_element_type=jnp.float32)

    return pl.pallas_call(
        body,
        out_shape=jax.ShapeDtypeStruct((B, SQ, E), jnp.float32),
        in_specs=[pl.BlockSpec(memory_space=pltpu.VMEM)] * 5,
        out_specs=pl.BlockSpec(memory_space=pltpu.VMEM),
        scratch_shapes=[
            pltpu.VMEM((B, SKV, D), jnp.float32),
            pltpu.VMEM((B, SKV, D), jnp.float32),
            pltpu.VMEM((2, B, SKV, D), jnp.float32),
            pltpu.VMEM((2, B, SKV, D), jnp.float32),
            pltpu.VMEM((2, B, SKV, D), jnp.float32),
            pltpu.VMEM((2, B, SKV, D), jnp.float32),
            pltpu.VMEM((B, SQ, D), jnp.float32),
            pltpu.VMEM((B, HQ, SQ, DH + 1), jnp.float32),
            pltpu.SemaphoreType.DMA((R_HOPS,)),
            pltpu.SemaphoreType.DMA((R_HOPS,)),
            pltpu.SemaphoreType.DMA((R_HOPS,)),
            pltpu.SemaphoreType.DMA((R_HOPS,)),
            pltpu.SemaphoreType.DMA((L_HOPS,)),
            pltpu.SemaphoreType.DMA((L_HOPS,)),
            pltpu.SemaphoreType.DMA((L_HOPS,)),
            pltpu.SemaphoreType.DMA((L_HOPS,)),
        ],
        compiler_params=pltpu.CompilerParams(collective_id=0),
    )(x, Wq, K_ext, V_ext, Wo)


# device time: 195809 ns/iter; 1.9893x vs baseline; 1.0659x over previous
import jax
import jax.numpy as jnp
from jax import lax
from jax.experimental import pallas as pl
from jax.experimental.pallas import tpu as pltpu

N_DEV = 8
BLK = 64
FULL_HOPS = 3
HOPS = 4


def kernel(x, Wq, K_ext, V_ext, Wo):
    B, SQ, E = x.shape
    _, SKV, HQ, DH = K_ext.shape
    D = HQ * DH
    HALF = SKV // 2

    def body(x_ref, wq_ref, k_ref, v_ref, wo_ref, out_ref,
             kl_ref, vl_ref, kcr_ref, vcr_ref, kcl_ref, vcl_ref,
             q_ref, acc_ref,
             ksr, krr, vsr, vrr, ksl, krl, vsl, vrl):
        my = lax.axis_index("i")
        left = lax.rem(my + N_DEV - 1, N_DEV)
        right = lax.rem(my + 1, N_DEV)

        barrier_sem = pltpu.get_barrier_semaphore()
        for nbr in (left, right):
            pl.semaphore_signal(
                barrier_sem, inc=1,
                device_id=(nbr,), device_id_type=pl.DeviceIdType.MESH,
            )
        pl.semaphore_wait(barrier_sem, 2)

        for b in range(B):
            kl_ref[b, :, :] = k_ref[b, :, :, :].reshape(SKV, D)
            vl_ref[b, :, :] = v_ref[b, :, :, :].reshape(SKV, D)

        kv_blocks = SKV // BLK

        def attend(origin, kget, vget, first, col_off=0, skv=SKV):
            qb = my * (SQ // BLK) + lax.broadcasted_iota(
                jnp.int32, (SQ, skv), 0) // BLK
            kb = origin * kv_blocks + (
                col_off + lax.broadcasted_iota(jnp.int32, (SQ, skv), 1)) // BLK
            mask = (qb == kb) | (kb == 0) | (lax.rem(qb + kb, 3) == 0)
            ones_col = jnp.ones((skv, 1), jnp.float32)
            for b in range(B):
                kcb = kget(b)
                vcb = vget(b)
                for hh in range(HQ):
                    qbh = q_ref[b, :, hh * DH:(hh + 1) * DH]
                    khh = kcb[:, hh * DH:(hh + 1) * DH]
                    s = lax.dot_general(
                        qbh, khh, (((1,), (1,)), ((), ())),
                        preferred_element_type=jnp.float32)
                    w = jnp.where(mask, jnp.exp(s), 0.0)
                    v_ext = jnp.concatenate(
                        [vcb[:, hh * DH:(hh + 1) * DH], ones_col], axis=1)
                    delta = jnp.dot(
                        w, v_ext, preferred_element_type=jnp.float32)
                    if first:
                        acc_ref[b, hh, :, :] = delta
                    else:
                        acc_ref[b, hh, :, :] = acc_ref[b, hh, :, :] + delta

        def rdma(src, dst, ssem, rsem, dev):
            return pltpu.make_async_remote_copy(
                src_ref=src, dst_ref=dst, send_sem=ssem, recv_sem=rsem,
                device_id=(dev,), device_id_type=pl.DeviceIdType.MESH,
            )

        for s in range(HOPS + 1):
            waits = []
            if s < FULL_HOPS:
                ksrc = kl_ref if s == 0 else kcr_ref.at[(s - 1) % 2]
                vsrc = vl_ref if s == 0 else vcr_ref.at[(s - 1) % 2]
                rr = [rdma(ksrc, kcr_ref.at[s % 2], ksr.at[s], krr.at[s], right),
                      rdma(vsrc, vcr_ref.at[s % 2], vsr.at[s], vrr.at[s], right)]
                lsrc_k = kl_ref if s == 0 else kcl_ref.at[(s - 1) % 2]
                lsrc_v = vl_ref if s == 0 else vcl_ref.at[(s - 1) % 2]
                ll = [rdma(lsrc_k, kcl_ref.at[s % 2], ksl.at[s], krl.at[s], left),
                      rdma(lsrc_v, vcl_ref.at[s % 2], vsl.at[s], vrl.at[s], left)]
                waits = rr + ll
            elif s == FULL_HOPS:
                slot = (s - 1) % 2
                dst = s % 2
                waits = [
                    rdma(kcr_ref.at[slot, :, 0:HALF, :],
                         kcr_ref.at[dst, :, 0:HALF, :],
                         ksr.at[s], krr.at[s], right),
                    rdma(vcr_ref.at[slot, :, 0:HALF, :],
                         vcr_ref.at[dst, :, 0:HALF, :],
                         vsr.at[s], vrr.at[s], right),
                    rdma(kcl_ref.at[slot, :, HALF:SKV, :],
                         kcl_ref.at[dst, :, HALF:SKV, :],
                         ksl.at[s], krl.at[s], left),
                    rdma(vcl_ref.at[slot, :, HALF:SKV, :],
                         vcl_ref.at[dst, :, HALF:SKV, :],
                         vsl.at[s], vrl.at[s], left),
                ]
            for r in waits:
                r.start()

            if s == 0:
                for b in range(B):
                    q_ref[b, :, :] = jnp.dot(
                        x_ref[b, :, :], wq_ref[:, :],
                        preferred_element_type=jnp.float32) * 0.125
                attend(my,
                       lambda b: kl_ref[b, :, :],
                       lambda b: vl_ref[b, :, :], first=True)
            elif s <= HOPS - 1:
                slot = (s - 1) % 2
                attend(lax.rem(my - s + N_DEV, N_DEV),
                       lambda b: kcr_ref[slot, b, :, :],
                       lambda b: vcr_ref[slot, b, :, :], first=False)
                attend(lax.rem(my + s, N_DEV),
                       lambda b: kcl_ref[slot, b, :, :],
                       lambda b: vcl_ref[slot, b, :, :], first=False)
            else:
                anti = lax.rem(my + HOPS, N_DEV)
                hslot = (HOPS - 1) % 2
                attend(anti,
                       lambda b: kcr_ref[hslot, b, 0:HALF, :],
                       lambda b: vcr_ref[hslot, b, 0:HALF, :], first=False,
                       col_off=0, skv=HALF)
                attend(anti,
                       lambda b: kcl_ref[hslot, b, HALF:SKV, :],
                       lambda b: vcl_ref[hslot, b, HALF:SKV, :], first=False,
                       col_off=HALF, skv=HALF)

            for r in waits:
                r.wait()

        for b in range(B):
            ctx = jnp.concatenate(
                [acc_ref[b, hh, :, 0:DH] / acc_ref[b, hh, :, DH:DH + 1]
                 for hh in range(HQ)], axis=1)
            out_ref[b, :, :] = jnp.dot(
                ctx, wo_ref[:, :], preferred_element_type=jnp.float32)

    return pl.pallas_call(
        body,
        out_shape=jax.ShapeDtypeStruct((B, SQ, E), jnp.float32),
        in_specs=[pl.BlockSpec(memory_space=pltpu.VMEM)] * 5,
        out_specs=pl.BlockSpec(memory_space=pltpu.VMEM),
        scratch_shapes=[
            pltpu.VMEM((B, SKV, D), jnp.float32),
            pltpu.VMEM((B, SKV, D), jnp.float32),
            pltpu.VMEM((2, B, SKV, D), jnp.float32),
            pltpu.VMEM((2, B, SKV, D), jnp.float32),
            pltpu.VMEM((2, B, SKV, D), jnp.float32),
            pltpu.VMEM((2, B, SKV, D), jnp.float32),
            pltpu.VMEM((B, SQ, D), jnp.float32),
            pltpu.VMEM((B, HQ, SQ, DH + 1), jnp.float32),
            pltpu.SemaphoreType.DMA((HOPS,)),
            pltpu.SemaphoreType.DMA((HOPS,)),
            pltpu.SemaphoreType.DMA((HOPS,)),
            pltpu.SemaphoreType.DMA((HOPS,)),
            pltpu.SemaphoreType.DMA((HOPS,)),
            pltpu.SemaphoreType.DMA((HOPS,)),
            pltpu.SemaphoreType.DMA((HOPS,)),
            pltpu.SemaphoreType.DMA((HOPS,)),
        ],
        compiler_params=pltpu.CompilerParams(
            collective_id=0, vmem_limit_bytes=48 * 1024 * 1024),
    )(x, Wq, K_ext, V_ext, Wo)


# device time: 116838 ns/iter; 3.3339x vs baseline; 1.6759x over previous
import jax
import jax.numpy as jnp
from jax import lax
from jax.experimental import pallas as pl
from jax.experimental.pallas import tpu as pltpu

N_DEV = 8
BLK = 64
FULL_HOPS = 3
HOPS = 4


def kernel(x, Wq, K_ext, V_ext, Wo):
    B, SQ, E = x.shape
    _, SKV, HQ, DH = K_ext.shape
    D = HQ * DH
    HALF = SKV // 2

    def body(x_ref, wq_ref, k_ref, v_ref, wo_ref, out_ref,
             kl_ref, vl_ref, kcr_ref, vcr_ref, kcl_ref, vcl_ref,
             q_ref, acc_ref,
             ksr, krr, vsr, vrr, ksl, krl, vsl, vrl):
        my = lax.axis_index("i")
        left = lax.rem(my + N_DEV - 1, N_DEV)
        right = lax.rem(my + 1, N_DEV)

        barrier_sem = pltpu.get_barrier_semaphore()
        for nbr in (left, right):
            pl.semaphore_signal(
                barrier_sem, inc=1,
                device_id=(nbr,), device_id_type=pl.DeviceIdType.MESH,
            )
        pl.semaphore_wait(barrier_sem, 2)

        for b in range(B):
            kl_ref[b, :, :] = k_ref[b, :, :, :].reshape(SKV, D).astype(
                jnp.bfloat16)
            vl_ref[b, :, :] = v_ref[b, :, :, :].reshape(SKV, D).astype(
                jnp.bfloat16)

        kv_blocks = SKV // BLK

        def attend(origin, kget, vget, first, col_off=0, skv=SKV):
            qb = my * (SQ // BLK) + lax.broadcasted_iota(
                jnp.int32, (SQ, skv), 0) // BLK
            kb = origin * kv_blocks + (
                col_off + lax.broadcasted_iota(jnp.int32, (SQ, skv), 1)) // BLK
            mask = (qb == kb) | (kb == 0) | (lax.rem(qb + kb, 3) == 0)
            ones_col = jnp.ones((skv, 1), jnp.bfloat16)
            for b in range(B):
                kcb = kget(b)
                vcb = vget(b)
                for hh in range(HQ):
                    qbh = q_ref[b, :, hh * DH:(hh + 1) * DH]
                    khh = kcb[:, hh * DH:(hh + 1) * DH]
                    s = lax.dot_general(
                        qbh, khh, (((1,), (1,)), ((), ())),
                        preferred_element_type=jnp.float32)
                    w = jnp.where(mask, jnp.exp(s), 0.0).astype(
                        jnp.bfloat16)
                    v_ext = jnp.concatenate(
                        [vcb[:, hh * DH:(hh + 1) * DH], ones_col], axis=1)
                    delta = jnp.dot(
                        w, v_ext, preferred_element_type=jnp.float32)
                    if first:
                        acc_ref[b, hh, :, :] = delta
                    else:
                        acc_ref[b, hh, :, :] = acc_ref[b, hh, :, :] + delta

        def rdma(src, dst, ssem, rsem, dev):
            return pltpu.make_async_remote_copy(
                src_ref=src, dst_ref=dst, send_sem=ssem, recv_sem=rsem,
                device_id=(dev,), device_id_type=pl.DeviceIdType.MESH,
            )

        for s in range(HOPS + 1):
            waits = []
            if s < FULL_HOPS:
                ksrc = kl_ref if s == 0 else kcr_ref.at[(s - 1) % 2]
                vsrc = vl_ref if s == 0 else vcr_ref.at[(s - 1) % 2]
                rr = [rdma(ksrc, kcr_ref.at[s % 2], ksr.at[s], krr.at[s], right),
                      rdma(vsrc, vcr_ref.at[s % 2], vsr.at[s], vrr.at[s], right)]
                lsrc_k = kl_ref if s == 0 else kcl_ref.at[(s - 1) % 2]
                lsrc_v = vl_ref if s == 0 else vcl_ref.at[(s - 1) % 2]
                ll = [rdma(lsrc_k, kcl_ref.at[s % 2], ksl.at[s], krl.at[s], left),
                      rdma(lsrc_v, vcl_ref.at[s % 2], vsl.at[s], vrl.at[s], left)]
                waits = rr + ll
            elif s == FULL_HOPS:
                slot = (s - 1) % 2
                dst = s % 2
                waits = [
                    rdma(kcr_ref.at[slot, :, 0:HALF, :],
                         kcr_ref.at[dst, :, 0:HALF, :],
                         ksr.at[s], krr.at[s], right),
                    rdma(vcr_ref.at[slot, :, 0:HALF, :],
                         vcr_ref.at[dst, :, 0:HALF, :],
                         vsr.at[s], vrr.at[s], right),
                    rdma(kcl_ref.at[slot, :, HALF:SKV, :],
                         kcl_ref.at[dst, :, HALF:SKV, :],
                         ksl.at[s], krl.at[s], left),
                    rdma(vcl_ref.at[slot, :, HALF:SKV, :],
                         vcl_ref.at[dst, :, HALF:SKV, :],
                         vsl.at[s], vrl.at[s], left),
                ]
            for r in waits:
                r.start()

            if s == 0:
                for b in range(B):
                    q_ref[b, :, :] = (jnp.dot(
                        x_ref[b, :, :].astype(jnp.bfloat16),
                        wq_ref[:, :].astype(jnp.bfloat16),
                        preferred_element_type=jnp.float32) * 0.125).astype(
                            jnp.bfloat16)
                attend(my,
                       lambda b: kl_ref[b, :, :],
                       lambda b: vl_ref[b, :, :], first=True)
            elif s <= HOPS - 1:
                slot = (s - 1) % 2
                attend(lax.rem(my - s + N_DEV, N_DEV),
                       lambda b: kcr_ref[slot, b, :, :],
                       lambda b: vcr_ref[slot, b, :, :], first=False)
                attend(lax.rem(my + s, N_DEV),
                       lambda b: kcl_ref[slot, b, :, :],
                       lambda b: vcl_ref[slot, b, :, :], first=False)
            else:
                anti = lax.rem(my + HOPS, N_DEV)
                hslot = (HOPS - 1) % 2
                attend(anti,
                       lambda b: kcr_ref[hslot, b, 0:HALF, :],
                       lambda b: vcr_ref[hslot, b, 0:HALF, :], first=False,
                       col_off=0, skv=HALF)
                attend(anti,
                       lambda b: kcl_ref[hslot, b, HALF:SKV, :],
                       lambda b: vcl_ref[hslot, b, HALF:SKV, :], first=False,
                       col_off=HALF, skv=HALF)

            for r in waits:
                r.wait()

        for b in range(B):
            ctx = jnp.concatenate(
                [acc_ref[b, hh, :, 0:DH] / acc_ref[b, hh, :, DH:DH + 1]
                 for hh in range(HQ)], axis=1).astype(jnp.bfloat16)
            out_ref[b, :, :] = jnp.dot(
                ctx, wo_ref[:, :].astype(jnp.bfloat16),
                preferred_element_type=jnp.float32)

    return pl.pallas_call(
        body,
        out_shape=jax.ShapeDtypeStruct((B, SQ, E), jnp.float32),
        in_specs=[pl.BlockSpec(memory_space=pltpu.VMEM)] * 5,
        out_specs=pl.BlockSpec(memory_space=pltpu.VMEM),
        scratch_shapes=[
            pltpu.VMEM((B, SKV, D), jnp.bfloat16),
            pltpu.VMEM((B, SKV, D), jnp.bfloat16),
            pltpu.VMEM((2, B, SKV, D), jnp.bfloat16),
            pltpu.VMEM((2, B, SKV, D), jnp.bfloat16),
            pltpu.VMEM((2, B, SKV, D), jnp.bfloat16),
            pltpu.VMEM((2, B, SKV, D), jnp.bfloat16),
            pltpu.VMEM((B, SQ, D), jnp.bfloat16),
            pltpu.VMEM((B, HQ, SQ, DH + 1), jnp.float32),
            pltpu.SemaphoreType.DMA((HOPS,)),
            pltpu.SemaphoreType.DMA((HOPS,)),
            pltpu.SemaphoreType.DMA((HOPS,)),
            pltpu.SemaphoreType.DMA((HOPS,)),
            pltpu.SemaphoreType.DMA((HOPS,)),
            pltpu.SemaphoreType.DMA((HOPS,)),
            pltpu.SemaphoreType.DMA((HOPS,)),
            pltpu.SemaphoreType.DMA((HOPS,)),
        ],
        compiler_params=pltpu.CompilerParams(
            collective_id=0, vmem_limit_bytes=48 * 1024 * 1024),
    )(x, Wq, K_ext, V_ext, Wo)


# device time: 116326 ns/iter; 3.3486x vs baseline; 1.0044x over previous
import jax
import jax.numpy as jnp
from jax import lax
from jax.experimental import pallas as pl
from jax.experimental.pallas import tpu as pltpu

N_DEV = 8
BLK = 64
FULL_HOPS = 3
HOPS = 4


def kernel(x, Wq, K_ext, V_ext, Wo):
    B, SQ, E = x.shape
    _, SKV, HQ, DH = K_ext.shape
    D = HQ * DH
    D2 = 2 * D
    HALF = SKV // 2

    def body(x_ref, wq_ref, k_ref, v_ref, wo_ref, out_ref,
             kvl_ref, cr_ref, cl_ref, q_ref, acc_ref,
             sr, rr, sl, rl):
        my = lax.axis_index("i")
        left = lax.rem(my + N_DEV - 1, N_DEV)
        right = lax.rem(my + 1, N_DEV)

        barrier_sem = pltpu.get_barrier_semaphore()
        for nbr in (left, right):
            pl.semaphore_signal(
                barrier_sem, inc=1,
                device_id=(nbr,), device_id_type=pl.DeviceIdType.MESH,
            )
        pl.semaphore_wait(barrier_sem, 2)

        for b in range(B):
            kvl_ref[b, :, 0:D] = k_ref[b, :, :, :].astype(
                jnp.bfloat16).reshape(SKV, D)
            kvl_ref[b, :, D:D2] = v_ref[b, :, :, :].astype(
                jnp.bfloat16).reshape(SKV, D)

        kv_blocks = SKV // BLK

        def attend(origin, kvget, first, col_off=0, skv=SKV):
            qb = my * (SQ // BLK) + lax.broadcasted_iota(
                jnp.int32, (SQ, skv), 0) // BLK
            kb = origin * kv_blocks + (
                col_off + lax.broadcasted_iota(jnp.int32, (SQ, skv), 1)) // BLK
            mask = (qb == kb) | (kb == 0) | (lax.rem(qb + kb, 3) == 0)
            ones_col = jnp.ones((skv, 1), jnp.bfloat16)
            for b in range(B):
                kvcb = kvget(b)
                for hh in range(HQ):
                    qbh = q_ref[b, :, hh * DH:(hh + 1) * DH]
                    khh = kvcb[:, hh * DH:(hh + 1) * DH]
                    s = lax.dot_general(
                        qbh, khh, (((1,), (1,)), ((), ())),
                        preferred_element_type=jnp.float32)
                    w = jnp.where(mask, jnp.exp(s.astype(jnp.bfloat16)), 0.0)
                    v_ext = jnp.concatenate(
                        [kvcb[:, D + hh * DH:D + (hh + 1) * DH], ones_col],
                        axis=1)
                    delta = jnp.dot(
                        w, v_ext, preferred_element_type=jnp.float32)
                    if first:
                        acc_ref[b, hh, :, :] = delta
                    else:
                        acc_ref[b, hh, :, :] = acc_ref[b, hh, :, :] + delta

        def rdma(src, dst, ssem, rsem, dev):
            return pltpu.make_async_remote_copy(
                src_ref=src, dst_ref=dst, send_sem=ssem, recv_sem=rsem,
                device_id=(dev,), device_id_type=pl.DeviceIdType.MESH,
            )

        for s in range(HOPS + 1):
            waits = []
            if s < FULL_HOPS:
                rsrc = kvl_ref if s == 0 else cr_ref.at[(s - 1) % 2]
                lsrc = kvl_ref if s == 0 else cl_ref.at[(s - 1) % 2]
                waits = [
                    rdma(rsrc, cr_ref.at[s % 2], sr.at[s], rr.at[s], right),
                    rdma(lsrc, cl_ref.at[s % 2], sl.at[s], rl.at[s], left),
                ]
            elif s == FULL_HOPS:
                slot = (s - 1) % 2
                dst = s % 2
                waits = [
                    rdma(cr_ref.at[slot, :, 0:HALF, :],
                         cr_ref.at[dst, :, 0:HALF, :],
                         sr.at[s], rr.at[s], right),
                    rdma(cl_ref.at[slot, :, HALF:SKV, :],
                         cl_ref.at[dst, :, HALF:SKV, :],
                         sl.at[s], rl.at[s], left),
                ]
            for r in waits:
                r.start()

            if s == 0:
                for b in range(B):
                    q_ref[b, :, :] = (jnp.dot(
                        x_ref[b, :, :].astype(jnp.bfloat16),
                        wq_ref[:, :].astype(jnp.bfloat16),
                        preferred_element_type=jnp.float32) * 0.125).astype(
                            jnp.bfloat16)
                attend(my, lambda b: kvl_ref[b, :, :], first=True)
            elif s <= HOPS - 1:
                slot = (s - 1) % 2
                attend(lax.rem(my - s + N_DEV, N_DEV),
                       lambda b: cr_ref[slot, b, :, :], first=False)
                attend(lax.rem(my + s, N_DEV),
                       lambda b: cl_ref[slot, b, :, :], first=False)
            else:
                anti = lax.rem(my + HOPS, N_DEV)
                hslot = (HOPS - 1) % 2
                attend(anti, lambda b: cr_ref[hslot, b, 0:HALF, :],
                       first=False, col_off=0, skv=HALF)
                attend(anti, lambda b: cl_ref[hslot, b, HALF:SKV, :],
                       first=False, col_off=HALF, skv=HALF)

            for r in waits:
                r.wait()

        for b in range(B):
            ctx = jnp.concatenate(
                [acc_ref[b, hh, :, 0:DH] / acc_ref[b, hh, :, DH:DH + 1]
                 for hh in range(HQ)], axis=1).astype(jnp.bfloat16)
            out_ref[b, :, :] = jnp.dot(
                ctx, wo_ref[:, :].astype(jnp.bfloat16),
                preferred_element_type=jnp.float32)

    return pl.pallas_call(
        body,
        out_shape=jax.ShapeDtypeStruct((B, SQ, E), jnp.float32),
        in_specs=[pl.BlockSpec(memory_space=pltpu.VMEM)] * 5,
        out_specs=pl.BlockSpec(memory_space=pltpu.VMEM),
        scratch_shapes=[
            pltpu.VMEM((B, SKV, D2), jnp.bfloat16),
            pltpu.VMEM((2, B, SKV, D2), jnp.bfloat16),
            pltpu.VMEM((2, B, SKV, D2), jnp.bfloat16),
            pltpu.VMEM((B, SQ, D), jnp.bfloat16),
            pltpu.VMEM((B, HQ, SQ, DH + 1), jnp.float32),
            pltpu.SemaphoreType.DMA((HOPS,)),
            pltpu.SemaphoreType.DMA((HOPS,)),
            pltpu.SemaphoreType.DMA((HOPS,)),
            pltpu.SemaphoreType.DMA((HOPS,)),
        ],
        compiler_params=pltpu.CompilerParams(
            collective_id=0, vmem_limit_bytes=48 * 1024 * 1024),
    )(x, Wq, K_ext, V_ext, Wo)
